# baseline (speedup 1.0000x reference)
import functools

import jax
import jax.numpy as jnp
from jax import lax
from jax.experimental import pallas as pl
from jax.experimental.pallas import tpu as pltpu
from jax.experimental.pallas import tpu_sc as plsc

K = 4

_NC = 2
_NS = 16
_NW = _NC * _NS


def _stats_kernel(x_ref, sum_ref, topk_ref):
    v = x_ref[...]
    total = jnp.sum(v, axis=(1, 2))
    sum_ref[...] = total[:, None]

    cb, r, lanes = v.shape
    g = r // 8

    def body(i, carry):
        a1, a2, a3, a4 = carry
        s = x_ref[:, pl.ds(i * 8, 8), :]
        t = jnp.maximum(a1, s); s = jnp.minimum(a1, s); a1 = t
        t = jnp.maximum(a2, s); s = jnp.minimum(a2, s); a2 = t
        t = jnp.maximum(a3, s); s = jnp.minimum(a3, s); a3 = t
        a4 = jnp.maximum(a4, s)
        return a1, a2, a3, a4

    neg = jnp.full((cb, 8, lanes), -jnp.inf, jnp.float32)
    a1, a2, a3, a4 = jax.lax.fori_loop(0, g, body, (neg, neg, neg, neg))
    cand = jnp.concatenate([a1, a2, a3, a4], axis=1)

    acc = jnp.zeros((cb,), jnp.float32)
    k_rem = jnp.full((cb,), float(K))
    for _ in range(K):
        m = jnp.max(cand, axis=(1, 2))
        eq = cand == m[:, None, None]
        cnt = jnp.sum(eq.astype(jnp.float32), axis=(1, 2))
        take = jnp.minimum(cnt, k_rem)
        acc = acc + jnp.where(take > 0, m * take, 0.0)
        k_rem = k_rem - take
        cand = jnp.where(eq, -jnp.inf, cand)
    topk_ref[...] = acc[:, None]


def _gate_kernel(sum_ref, topk_ref, w1_ref, b1_ref, w2_ref, b2_ref, gate_ref,
                 gate16_ref, *, inv_n):
    avg = sum_ref[...] * inv_n
    tk = topk_ref[...]

    def fc(v):
        h = jnp.dot(w1_ref[...], v, preferred_element_type=jnp.float32)
        h = jnp.maximum(h + b1_ref[...], 0.0)
        o = jnp.dot(w2_ref[...], h, preferred_element_type=jnp.float32)
        return o + b2_ref[...]

    score = fc(avg) + fc(tk)
    gate = jax.nn.sigmoid(score)
    gate_ref[...] = gate
    gate16_ref[...] = jnp.broadcast_to(gate, gate16_ref.shape)


def _sc_scale_body(x_hbm, gate_hbm, out_hbm, gate_v, buf, *, n, cpw, chunk):
    w = lax.axis_index("s") * _NC + lax.axis_index("c")
    nchunk = n // chunk
    nv = chunk // 256

    def ch_body(cc, _):
        ch = w * cpw + cc
        pltpu.sync_copy(gate_hbm.at[pl.ds(ch * 16, 16)], gate_v)
        g = gate_v[...]

        def chunk_body(kk, _):
            off = ch * n + kk * chunk
            pltpu.sync_copy(x_hbm.at[pl.ds(off, chunk)], buf)

            def vbody(i, _):
                base = i * 256
                for j in range(16):
                    s = pl.ds(base + j * 16, 16)
                    buf[s] = buf[s] * g
                return 0

            lax.fori_loop(0, nv, vbody, 0)
            pltpu.sync_copy(buf, out_hbm.at[pl.ds(off, chunk)])
            return 0

        lax.fori_loop(0, nchunk, chunk_body, 0)
        return 0

    lax.fori_loop(0, cpw, ch_body, 0)


def kernel(x, W1, b1, W2, b2):
    b, c, d, h, w = x.shape
    n = d * h * w
    assert b == 1
    lanes = 128
    rows = n // lanes
    xr = x.reshape(c, rows, lanes)

    cb = 8
    grid = c // cb

    sums, topks = pl.pallas_call(
        _stats_kernel,
        grid=(grid,),
        in_specs=[pl.BlockSpec((cb, rows, lanes), lambda i: (i, 0, 0))],
        out_specs=[
            pl.BlockSpec((cb, 1), lambda i: (i, 0)),
            pl.BlockSpec((cb, 1), lambda i: (i, 0)),
        ],
        out_shape=[
            jax.ShapeDtypeStruct((c, 1), jnp.float32),
            jax.ShapeDtypeStruct((c, 1), jnp.float32),
        ],
        compiler_params=pltpu.CompilerParams(
            dimension_semantics=("parallel",)),
    )(xr)

    gate, gate16 = pl.pallas_call(
        functools.partial(_gate_kernel, inv_n=1.0 / n),
        out_shape=[
            jax.ShapeDtypeStruct((c, 1), jnp.float32),
            jax.ShapeDtypeStruct((c, 16), jnp.float32),
        ],
    )(sums, topks, W1, b1[:, None], W2, b2[:, None])

    assert c % _NW == 0
    chunk = 25088
    assert n % chunk == 0

    y1 = pl.kernel(
        functools.partial(_sc_scale_body, n=n, cpw=c // _NW, chunk=chunk),
        out_type=jax.ShapeDtypeStruct((c * n,), jnp.float32),
        mesh=plsc.VectorSubcoreMesh(core_axis_name="c", subcore_axis_name="s"),
        scratch_types=[
            pltpu.VMEM((16,), jnp.float32),
            pltpu.VMEM((chunk,), jnp.float32),
        ],
    )(x.reshape(c * n), gate16.reshape(c * 16))

    out = gate.reshape(b, c, 1, 1, 1)
    return (y1.reshape(b, c, d, h, w), out)


# speedup vs baseline: 1.0826x; 1.0826x over previous
import functools

import jax
import jax.numpy as jnp
from jax import lax
from jax.experimental import pallas as pl
from jax.experimental.pallas import tpu as pltpu
from jax.experimental.pallas import tpu_sc as plsc

K = 4

_NC = 2
_NS = 16
_NW = _NC * _NS


def _stats_kernel(x_ref, sum_ref, topk_ref):
    v = x_ref[...]
    total = jnp.sum(v, axis=(1, 2))
    sum_ref[...] = total[:, None]

    cb, r, lanes = v.shape
    g = r // 8

    def body(i, carry):
        a1, a2, a3, a4 = carry
        s = x_ref[:, pl.ds(i * 8, 8), :]
        t = jnp.maximum(a1, s); s = jnp.minimum(a1, s); a1 = t
        t = jnp.maximum(a2, s); s = jnp.minimum(a2, s); a2 = t
        t = jnp.maximum(a3, s); s = jnp.minimum(a3, s); a3 = t
        a4 = jnp.maximum(a4, s)
        return a1, a2, a3, a4

    neg = jnp.full((cb, 8, lanes), -jnp.inf, jnp.float32)
    a1, a2, a3, a4 = jax.lax.fori_loop(0, g, body, (neg, neg, neg, neg))
    cand = jnp.concatenate([a1, a2, a3, a4], axis=1)

    acc = jnp.zeros((cb,), jnp.float32)
    k_rem = jnp.full((cb,), float(K))
    for _ in range(K):
        m = jnp.max(cand, axis=(1, 2))
        eq = cand == m[:, None, None]
        cnt = jnp.sum(eq.astype(jnp.float32), axis=(1, 2))
        take = jnp.minimum(cnt, k_rem)
        acc = acc + jnp.where(take > 0, m * take, 0.0)
        k_rem = k_rem - take
        cand = jnp.where(eq, -jnp.inf, cand)
    topk_ref[...] = acc[:, None]


def _gate_kernel(sum_ref, topk_ref, w1_ref, b1_ref, w2_ref, b2_ref, gate_ref,
                 gate16_ref, *, inv_n):
    avg = sum_ref[...] * inv_n
    tk = topk_ref[...]

    def fc(v):
        h = jnp.dot(w1_ref[...], v, preferred_element_type=jnp.float32)
        h = jnp.maximum(h + b1_ref[...], 0.0)
        o = jnp.dot(w2_ref[...], h, preferred_element_type=jnp.float32)
        return o + b2_ref[...]

    score = fc(avg) + fc(tk)
    gate = jax.nn.sigmoid(score)
    gate_ref[...] = gate
    gate16_ref[...] = jnp.broadcast_to(gate, gate16_ref.shape)


_NBUF = 4
_LOOKAHEAD = 2


def _sc_scale_body(x_hbm, gate_hbm, out_hbm, gate_v,
                   b0, b1, b2, b3, si0, si1, si2, si3, so0, so1, so2, so3,
                   *, n, cpw, chunk):
    w = lax.axis_index("s") * _NC + lax.axis_index("c")
    nchunk = n // chunk
    ntask = cpw * nchunk
    nv = chunk // 256
    bufs = [b0, b1, b2, b3]
    sin = [si0, si1, si2, si3]
    sout = [so0, so1, so2, so3]

    pltpu.sync_copy(gate_hbm.at[pl.ds(w * (cpw * 16), cpw * 16)], gate_v)

    def off(t):
        ch = w * cpw + t // nchunk
        return ch * n + (t % nchunk) * chunk

    hin = {}
    hout = {}
    for t in range(min(_LOOKAHEAD, ntask)):
        hin[t] = pltpu.async_copy(
            x_hbm.at[pl.ds(off(t), chunk)], bufs[t % _NBUF], sin[t % _NBUF])

    for t in range(ntask):
        b = t % _NBUF
        hin[t].wait()
        g = gate_v[pl.ds((t // nchunk) * 16, 16)]
        buf = bufs[b]

        def vbody(i, _, buf=buf, g=g):
            base = i * 256
            for j in range(16):
                s = pl.ds(base + j * 16, 16)
                buf[s] = buf[s] * g
            return 0

        lax.fori_loop(0, nv, vbody, 0)
        hout[t] = pltpu.async_copy(
            buf, out_hbm.at[pl.ds(off(t), chunk)], sout[b])
        nt = t + _LOOKAHEAD
        if nt < ntask:
            if nt - _NBUF >= 0:
                hout[nt - _NBUF].wait()
            hin[nt] = pltpu.async_copy(
                x_hbm.at[pl.ds(off(nt), chunk)],
                bufs[nt % _NBUF], sin[nt % _NBUF])
    for t in range(max(0, ntask - _NBUF), ntask):
        hout[t].wait()


def kernel(x, W1, b1, W2, b2):
    b, c, d, h, w = x.shape
    n = d * h * w
    assert b == 1
    lanes = 128
    rows = n // lanes
    xr = x.reshape(c, rows, lanes)

    cb = 8
    grid = c // cb

    sums, topks = pl.pallas_call(
        _stats_kernel,
        grid=(grid,),
        in_specs=[pl.BlockSpec((cb, rows, lanes), lambda i: (i, 0, 0))],
        out_specs=[
            pl.BlockSpec((cb, 1), lambda i: (i, 0)),
            pl.BlockSpec((cb, 1), lambda i: (i, 0)),
        ],
        out_shape=[
            jax.ShapeDtypeStruct((c, 1), jnp.float32),
            jax.ShapeDtypeStruct((c, 1), jnp.float32),
        ],
        compiler_params=pltpu.CompilerParams(
            dimension_semantics=("parallel",)),
    )(xr)

    gate, gate16 = pl.pallas_call(
        functools.partial(_gate_kernel, inv_n=1.0 / n),
        out_shape=[
            jax.ShapeDtypeStruct((c, 1), jnp.float32),
            jax.ShapeDtypeStruct((c, 16), jnp.float32),
        ],
    )(sums, topks, W1, b1[:, None], W2, b2[:, None])

    assert c % _NW == 0
    chunk = 25088
    assert n % chunk == 0

    y1 = pl.kernel(
        functools.partial(_sc_scale_body, n=n, cpw=c // _NW, chunk=chunk),
        out_type=jax.ShapeDtypeStruct((c * n,), jnp.float32),
        mesh=plsc.VectorSubcoreMesh(core_axis_name="c", subcore_axis_name="s"),
        scratch_types=(
            [pltpu.VMEM((c // _NW * 16,), jnp.float32)]
            + [pltpu.VMEM((chunk,), jnp.float32) for _ in range(_NBUF)]
            + [pltpu.SemaphoreType.DMA for _ in range(2 * _NBUF)]
        ),
    )(x.reshape(c * n), gate16.reshape(c * 16))

    out = gate.reshape(b, c, 1, 1, 1)
    return (y1.reshape(b, c, d, h, w), out)


# speedup vs baseline: 1.5197x; 1.4037x over previous
import functools

import jax
import jax.numpy as jnp
from jax.experimental import pallas as pl
from jax.experimental.pallas import tpu as pltpu

K = 4


def _fused_kernel(x_ref, w1_ref, b1_ref, w2_ref, b2_ref, y_ref, gate_ref,
                  cache_ref, sum_ref, topk_ref, gatev_ref, *,
                  grid_g, ccb, inv_n):
    i = pl.program_id(0)
    G = grid_g
    U = G - ccb
    cb, rows, lanes = x_ref.shape

    @pl.when(i < G)
    def _phase1():
        v = x_ref[...]

        @pl.when(i < ccb)
        def _store():
            cache_ref[pl.ds(i * cb, cb), :, :] = v

        total = jnp.sum(v, axis=(1, 2))
        sum_ref[pl.ds(i * cb, cb), :] = total[:, None]

        g = rows // 8

        def body(s_i, carry):
            a1, a2, a3, a4 = carry
            s = x_ref[:, pl.ds(s_i * 8, 8), :]
            t = jnp.maximum(a1, s); s = jnp.minimum(a1, s); a1 = t
            t = jnp.maximum(a2, s); s = jnp.minimum(a2, s); a2 = t
            t = jnp.maximum(a3, s); s = jnp.minimum(a3, s); a3 = t
            a4 = jnp.maximum(a4, s)
            return a1, a2, a3, a4

        neg = jnp.full((cb, 8, lanes), -jnp.inf, jnp.float32)
        a1, a2, a3, a4 = jax.lax.fori_loop(0, g, body, (neg, neg, neg, neg))
        cand = jnp.concatenate([a1, a2, a3, a4], axis=1)

        acc = jnp.zeros((cb,), jnp.float32)
        k_rem = jnp.full((cb,), float(K))
        for _ in range(K):
            m = jnp.max(cand, axis=(1, 2))
            eq = cand == m[:, None, None]
            cnt = jnp.sum(eq.astype(jnp.float32), axis=(1, 2))
            take = jnp.minimum(cnt, k_rem)
            acc = acc + jnp.where(take > 0, m * take, 0.0)
            k_rem = k_rem - take
            cand = jnp.where(eq, -jnp.inf, cand)
        topk_ref[pl.ds(i * cb, cb), :] = acc[:, None]

    @pl.when(i == G)
    def _gate():
        avg = sum_ref[...] * inv_n
        tk = topk_ref[...]

        def fc(v):
            h = jnp.dot(w1_ref[...], v, preferred_element_type=jnp.float32)
            h = jnp.maximum(h + b1_ref[...], 0.0)
            o = jnp.dot(w2_ref[...], h, preferred_element_type=jnp.float32)
            return o + b2_ref[...]

        gate = jax.nn.sigmoid(fc(avg) + fc(tk))
        gatev_ref[...] = gate
        gate_ref[...] = gate

    @pl.when(i >= G)
    def _phase2():
        j = i - G

        @pl.when(j < U)
        def _uncached():
            gv = gatev_ref[pl.ds((ccb + j) * cb, cb), :]
            y_ref[...] = x_ref[...] * gv[:, :, None]

        @pl.when(j >= U)
        def _cached():
            blk = j - U
            gv = gatev_ref[pl.ds(blk * cb, cb), :]
            y_ref[...] = cache_ref[pl.ds(blk * cb, cb), :, :] * gv[:, :, None]


def kernel(x, W1, b1, W2, b2):
    b, c, d, h, w = x.shape
    n = d * h * w
    assert b == 1
    lanes = 128
    rows = n // lanes
    xr = x.reshape(c, rows, lanes)

    cb = 4
    G = c // cb
    cc = 60
    ccb = cc // cb
    U = G - ccb

    def x_map(i):
        return (jnp.where(i < G, i, jnp.minimum(ccb + i - G, G - 1)), 0, 0)

    def y_map(i):
        j = i - G
        return (jnp.where(j < U, ccb + jnp.maximum(j, 0), j - U), 0, 0)

    y, gate = pl.pallas_call(
        functools.partial(_fused_kernel, grid_g=G, ccb=ccb, inv_n=1.0 / n),
        grid=(2 * G,),
        in_specs=[
            pl.BlockSpec((cb, rows, lanes), x_map),
            pl.BlockSpec((c // 2, c), lambda i: (0, 0)),
            pl.BlockSpec((c // 2, 1), lambda i: (0, 0)),
            pl.BlockSpec((c, c // 2), lambda i: (0, 0)),
            pl.BlockSpec((c, 1), lambda i: (0, 0)),
        ],
        out_specs=[
            pl.BlockSpec((cb, rows, lanes), y_map),
            pl.BlockSpec((c, 1), lambda i: (0, 0)),
        ],
        out_shape=[
            jax.ShapeDtypeStruct((c, rows, lanes), jnp.float32),
            jax.ShapeDtypeStruct((c, 1), jnp.float32),
        ],
        scratch_shapes=[
            pltpu.VMEM((cc, rows, lanes), jnp.float32),
            pltpu.VMEM((c, 1), jnp.float32),
            pltpu.VMEM((c, 1), jnp.float32),
            pltpu.VMEM((c, 1), jnp.float32),
        ],
        compiler_params=pltpu.CompilerParams(
            dimension_semantics=("arbitrary",),
            vmem_limit_bytes=64 * 1024 * 1024,
        ),
    )(xr, W1, b1[:, None], W2, b2[:, None])

    out = gate.reshape(b, c, 1, 1, 1)
    return (y.reshape(b, c, d, h, w), out)


# speedup vs baseline: 1.5438x; 1.0159x over previous
import functools

import jax
import jax.numpy as jnp
from jax.experimental import pallas as pl
from jax.experimental.pallas import tpu as pltpu

K = 4
_NBUF = 4
_NOB = 4


def _stats_block(buf, sum_ref, topk_ref, t, cb):
    v = buf[...]
    total = jnp.sum(v, axis=(1, 2))
    sum_ref[pl.ds(t * cb, cb), :] = total[:, None]

    _, rows, lanes = v.shape
    g = rows // 8

    def body(s_i, carry):
        a1, a2, a3, a4 = carry
        s = buf[:, pl.ds(s_i * 8, 8), :]
        t_ = jnp.maximum(a1, s); s = jnp.minimum(a1, s); a1 = t_
        t_ = jnp.maximum(a2, s); s = jnp.minimum(a2, s); a2 = t_
        t_ = jnp.maximum(a3, s); s = jnp.minimum(a3, s); a3 = t_
        a4 = jnp.maximum(a4, s)
        return a1, a2, a3, a4

    neg = jnp.full((cb, 8, lanes), -jnp.inf, jnp.float32)
    a1, a2, a3, a4 = jax.lax.fori_loop(0, g, body, (neg, neg, neg, neg))
    cand = jnp.concatenate([a1, a2, a3, a4], axis=1)

    acc = jnp.zeros((cb,), jnp.float32)
    k_rem = jnp.full((cb,), float(K))
    for _ in range(K):
        m = jnp.max(cand, axis=(1, 2))
        eq = cand == m[:, None, None]
        cnt = jnp.sum(eq.astype(jnp.float32), axis=(1, 2))
        take = jnp.minimum(cnt, k_rem)
        acc = acc + jnp.where(take > 0, m * take, 0.0)
        k_rem = k_rem - take
        cand = jnp.where(eq, -jnp.inf, cand)
    topk_ref[pl.ds(t * cb, cb), :] = acc[:, None]


def _ring_kernel(x_hbm, w1_ref, b1_ref, w2_ref, b2_ref, y_hbm, gate_ref,
                 *refs, nblk, cb, inv_n):
    bufs = refs[:_NBUF]
    obufs = refs[_NBUF:_NBUF + _NOB]
    sum_ref, topk_ref, gatev_ref = refs[_NBUF + _NOB:_NBUF + _NOB + 3]
    rsems = refs[_NBUF + _NOB + 3:_NBUF + _NOB + 3 + _NBUF]
    wsems = refs[_NBUF + _NOB + 3 + _NBUF:]

    def rcopy(t):
        return pltpu.make_async_copy(
            x_hbm.at[pl.ds(t * cb, cb)], bufs[t % _NBUF], rsems[t % _NBUF])

    def wcopy(t):
        return pltpu.make_async_copy(
            obufs[t % _NOB], y_hbm.at[pl.ds(t * cb, cb)], wsems[t % _NOB])

    for t in range(min(_NBUF, nblk)):
        rcopy(t).start()
    for t in range(nblk):
        rcopy(t).wait()
        _stats_block(bufs[t % _NBUF], sum_ref, topk_ref, t, cb)
        nt = t + _NBUF
        if nt < nblk:
            rcopy(nt).start()

    for t in range(min(_NBUF, nblk)):
        rcopy(t).start()

    avg = sum_ref[...] * inv_n
    tk = topk_ref[...]

    def fc(v):
        h = jnp.dot(w1_ref[...], v, preferred_element_type=jnp.float32)
        h = jnp.maximum(h + b1_ref[...], 0.0)
        o = jnp.dot(w2_ref[...], h, preferred_element_type=jnp.float32)
        return o + b2_ref[...]

    gate = jax.nn.sigmoid(fc(avg) + fc(tk))
    gatev_ref[...] = gate
    gate_ref[...] = gate

    for t in range(nblk):
        rcopy(t).wait()
        if t - _NOB >= 0:
            wcopy(t - _NOB).wait()
        gv = gatev_ref[pl.ds(t * cb, cb), :]
        obufs[t % _NOB][...] = bufs[t % _NBUF][...] * gv[:, :, None]
        wcopy(t).start()
        nt = t + _NBUF
        if nt < nblk:
            rcopy(nt).start()
    for t in range(max(0, nblk - _NOB), nblk):
        wcopy(t).wait()


def kernel(x, W1, b1, W2, b2):
    b, c, d, h, w = x.shape
    n = d * h * w
    assert b == 1
    lanes = 128
    rows = n // lanes
    xr = x.reshape(c, rows, lanes)

    cb = 8
    nblk = c // cb

    y, gate = pl.pallas_call(
        functools.partial(_ring_kernel, nblk=nblk, cb=cb, inv_n=1.0 / n),
        in_specs=[
            pl.BlockSpec(memory_space=pl.ANY),
            pl.BlockSpec((c // 2, c), lambda: (0, 0)),
            pl.BlockSpec((c // 2, 1), lambda: (0, 0)),
            pl.BlockSpec((c, c // 2), lambda: (0, 0)),
            pl.BlockSpec((c, 1), lambda: (0, 0)),
        ],
        out_specs=[
            pl.BlockSpec(memory_space=pl.ANY),
            pl.BlockSpec((c, 1), lambda: (0, 0)),
        ],
        out_shape=[
            jax.ShapeDtypeStruct((c, rows, lanes), jnp.float32),
            jax.ShapeDtypeStruct((c, 1), jnp.float32),
        ],
        scratch_shapes=(
            [pltpu.VMEM((cb, rows, lanes), jnp.float32)
             for _ in range(_NBUF + _NOB)]
            + [pltpu.VMEM((c, 1), jnp.float32) for _ in range(3)]
            + [pltpu.SemaphoreType.DMA for _ in range(_NBUF + _NOB)]
        ),
        compiler_params=pltpu.CompilerParams(
            vmem_limit_bytes=64 * 1024 * 1024,
        ),
    )(xr, W1, b1[:, None], W2, b2[:, None])

    out = gate.reshape(b, c, 1, 1, 1)
    return (y.reshape(b, c, d, h, w), out)
